# baseline (speedup 1.0000x reference)
import functools

import jax
import jax.numpy as jnp
from jax import lax
from jax.experimental import pallas as pl
from jax.experimental.pallas import tpu as pltpu

FEAT = 512
LATENT = 256
CF = 4
NCB = 4
NEMB = 1024
CDIM = 256
ALPHA = -5.0
B = 32
T = 800
ROWS = (B * T) // NCB
TILE = 640
NT = ROWS // TILE
BIG = 2 ** 30


def _body(xf_ref, w1t_ref, et_ref, e_ref, e2_ref, w2t_ref,
          out_ref, kidx_ref, commit_ref, ppls_ref,
          counts_ref, acc_ref):
    i = pl.program_id(0)
    r = pl.program_id(1)

    rows = xf_ref[0]
    z = lax.dot_general(rows.astype(jnp.bfloat16), w1t_ref[...],
                        (((1,), (0,)), ((), ())),
                        preferred_element_type=jnp.float32)
    xe = lax.dot_general(z.astype(jnp.bfloat16), et_ref[0],
                         (((1,), (0,)), ((), ())),
                         preferred_element_type=jnp.float32)
    x2 = jnp.sum(z * z, axis=1, keepdims=True)
    d = e2_ref[0] + x2 - 2.0 * xe
    dm = ALPHA * d
    m = jnp.max(dm, axis=1, keepdims=True)
    iota = lax.broadcasted_iota(jnp.int32, (TILE, NEMB), 1)
    k2 = jnp.min(jnp.where(dm == m, iota, BIG), axis=1, keepdims=True)

    hard = (iota == k2)
    q = lax.dot_general(hard.astype(jnp.bfloat16), e_ref[0],
                        (((1,), (0,)), ((), ())),
                        preferred_element_type=jnp.float32)
    out_ref[0] = lax.dot_general(q.astype(jnp.bfloat16), w2t_ref[...],
                                 (((1,), (0,)), ((), ())),
                                 preferred_element_type=jnp.float32)
    kidx_ref[...] = jnp.transpose(k2).reshape(1, 1, 1, TILE)

    cnt = jnp.sum(hard.astype(jnp.float32), axis=0, keepdims=True)

    @pl.when(r == 0)
    def _():
        counts_ref[...] = cnt

    @pl.when(r > 0)
    def _():
        counts_ref[...] = counts_ref[...] + cnt

    part = jnp.sum((z - q) ** 2)

    @pl.when((i == 0) & (r == 0))
    def _():
        acc_ref[0] = part

    @pl.when((i > 0) | (r > 0))
    def _():
        acc_ref[0] = acc_ref[0] + part

    @pl.when((i == NCB - 1) & (r == NT - 1))
    def _():
        commit_ref[...] = jnp.full((1, 1), acc_ref[0] / (ROWS * CDIM),
                                   jnp.float32)

    @pl.when(r == NT - 1)
    def _():
        probs = counts_ref[...] / float(ROWS)
        ent = -jnp.sum(probs * jnp.log2(probs + 1e-10))
        lane4 = lax.broadcasted_iota(jnp.int32, (1, NCB), 1)
        contrib = jnp.where(lane4 == i, ent, 0.0)
        prev = jnp.where(i == 0, jnp.zeros((1, NCB), jnp.float32),
                         ppls_ref[...])
        ppls_ref[...] = prev + contrib


@jax.jit
def _run(xf, w1t, et, e, e2, w2t):
    return pl.pallas_call(
        _body,
        grid=(NCB, NT),
        in_specs=[
            pl.BlockSpec((1, TILE, FEAT), lambda i, r: (i, r, 0)),
            pl.BlockSpec((FEAT, LATENT), lambda i, r: (0, 0)),
            pl.BlockSpec((1, CDIM, NEMB), lambda i, r: (i, 0, 0)),
            pl.BlockSpec((1, NEMB, CDIM), lambda i, r: (i, 0, 0)),
            pl.BlockSpec((1, 1, NEMB), lambda i, r: (i, 0, 0)),
            pl.BlockSpec((CDIM, FEAT), lambda i, r: (0, 0)),
        ],
        out_specs=[
            pl.BlockSpec((1, TILE, FEAT), lambda i, r: (i, r, 0)),
            pl.BlockSpec((1, 1, 1, TILE), lambda i, r: (i, r, 0, 0)),
            pl.BlockSpec((1, 1), lambda i, r: (0, 0)),
            pl.BlockSpec((1, NCB), lambda i, r: (0, 0)),
        ],
        out_shape=[
            jax.ShapeDtypeStruct((NCB, NCB * ROWS // NCB, FEAT), jnp.float32),
            jax.ShapeDtypeStruct((NCB, NT, 1, TILE), jnp.int32),
            jax.ShapeDtypeStruct((1, 1), jnp.float32),
            jax.ShapeDtypeStruct((1, NCB), jnp.float32),
        ],
        scratch_shapes=[
            pltpu.VMEM((1, NEMB), jnp.float32),
            pltpu.SMEM((1,), jnp.float32),
        ],
        compiler_params=pltpu.CompilerParams(
            dimension_semantics=("arbitrary", "arbitrary")),
    )(xf, w1t, et, e, e2, w2t)


def kernel(inputs, W1, W2, embeddings):
    b, c, t, _ = inputs.shape
    xf = jnp.transpose(inputs[..., 0].reshape(b, c, t // CF, CF),
                       (3, 0, 2, 1)).reshape(NCB, ROWS, FEAT)
    w1t = W1.T.astype(jnp.bfloat16)
    et = jnp.transpose(embeddings, (0, 2, 1)).astype(jnp.bfloat16)
    e = embeddings.astype(jnp.bfloat16)
    e2 = jnp.sum(embeddings ** 2, axis=2)[:, None, :]
    w2t = W2.T.astype(jnp.bfloat16)

    outf, kidx, commit, ppls = _run(xf, w1t, et, e, e2, w2t)

    out = jnp.transpose(outf.reshape(NCB, b, t // CF, FEAT),
                        (1, 3, 2, 0)).reshape(b, FEAT, t)[..., None]
    inds = kidx.reshape(NCB, b, t // CF)
    return out, commit[0, 0], ppls[0], inds


# speedup vs baseline: 1.8120x; 1.8120x over previous
import jax
import jax.numpy as jnp
from jax import lax
from jax.experimental import pallas as pl
from jax.experimental.pallas import tpu as pltpu

FEAT = 512
LATENT = 256
CF = 4
NCB = 4
NEMB = 1024
CDIM = 256
ALPHA = -5.0
B = 32
T = 800
ROWS = (B * T) // NCB
TILE = 640
NT = ROWS // TILE
BIG = 2 ** 30


def _stage_a(x_ref, w1_ref, z_ref):
    zt = lax.dot_general(w1_ref[...], x_ref[0].astype(jnp.bfloat16),
                         (((1,), (0,)), ((), ())),
                         preferred_element_type=jnp.float32)
    z_ref[0] = jnp.transpose(zt)


def _stage_b(z_ref, et_ref, e_ref, e2_ref,
             q_ref, kidx_ref, commit_ref, ppls_ref,
             counts_ref, acc_ref):
    i = pl.program_id(0)
    r = pl.program_id(1)

    z = z_ref[...]
    xe = lax.dot_general(z.astype(jnp.bfloat16), et_ref[0],
                         (((1,), (0,)), ((), ())),
                         preferred_element_type=jnp.float32)
    x2 = jnp.sum(z * z, axis=1, keepdims=True)
    d = e2_ref[0] + x2 - 2.0 * xe
    dm = ALPHA * d
    m = jnp.max(dm, axis=1, keepdims=True)
    iota = lax.broadcasted_iota(jnp.int32, (TILE, NEMB), 1)
    k2 = jnp.min(jnp.where(dm == m, iota, BIG), axis=1, keepdims=True)

    hard = (iota == k2).astype(jnp.bfloat16)
    q = lax.dot_general(hard, e_ref[0],
                        (((1,), (0,)), ((), ())),
                        preferred_element_type=jnp.float32)
    q_ref[...] = q
    kidx_ref[...] = jnp.transpose(k2).reshape(1, 1, 1, TILE)

    ones = jnp.full((8, TILE), jnp.bfloat16(1))
    cnt8 = lax.dot_general(ones, hard, (((1,), (0,)), ((), ())),
                           preferred_element_type=jnp.float32)
    cnt = cnt8[0:1]

    @pl.when(r == 0)
    def _():
        counts_ref[...] = cnt

    @pl.when(r > 0)
    def _():
        counts_ref[...] = counts_ref[...] + cnt

    part = jnp.sum((z - q) ** 2)

    @pl.when((i == 0) & (r == 0))
    def _():
        acc_ref[0] = part

    @pl.when((i > 0) | (r > 0))
    def _():
        acc_ref[0] = acc_ref[0] + part

    @pl.when((i == NCB - 1) & (r == NT - 1))
    def _():
        commit_ref[...] = jnp.full((1, 1), acc_ref[0] / (ROWS * CDIM),
                                   jnp.float32)

    @pl.when(r == NT - 1)
    def _():
        probs = counts_ref[...] / float(ROWS)
        ent = -jnp.sum(probs * jnp.log2(probs + 1e-10))
        lane4 = lax.broadcasted_iota(jnp.int32, (1, NCB), 1)
        contrib = jnp.where(lane4 == i, ent, 0.0)
        prev = jnp.where(i == 0, jnp.zeros((1, NCB), jnp.float32),
                         ppls_ref[...])
        ppls_ref[...] = prev + contrib


def _stage_c(q_ref, w2_ref, out_ref):
    qt = jnp.transpose(q_ref[0]).astype(jnp.bfloat16)
    out_ref[0] = lax.dot_general(w2_ref[...], qt,
                                 (((1,), (0,)), ((), ())),
                                 preferred_element_type=jnp.float32)


@jax.jit
def _run(x, W1, W2, embeddings):
    w1 = W1.astype(jnp.bfloat16)
    w2 = W2.astype(jnp.bfloat16)
    et = jnp.transpose(embeddings, (0, 2, 1)).astype(jnp.bfloat16)
    e = embeddings.astype(jnp.bfloat16)
    e2 = jnp.sum(embeddings ** 2, axis=2)[:, None, :]

    z = pl.pallas_call(
        _stage_a,
        grid=(B,),
        in_specs=[
            pl.BlockSpec((1, FEAT, T), lambda b: (b, 0, 0)),
            pl.BlockSpec((LATENT, FEAT), lambda b: (0, 0)),
        ],
        out_specs=pl.BlockSpec((1, T, LATENT), lambda b: (b, 0, 0)),
        out_shape=jax.ShapeDtypeStruct((B, T, LATENT), jnp.float32),
        compiler_params=pltpu.CompilerParams(
            dimension_semantics=("arbitrary",)),
    )(x, w1)

    zf = z.reshape(ROWS, NCB * CDIM)

    qf, kidx, commit, ppls = pl.pallas_call(
        _stage_b,
        grid=(NCB, NT),
        in_specs=[
            pl.BlockSpec((TILE, CDIM), lambda i, r: (r, i)),
            pl.BlockSpec((1, CDIM, NEMB), lambda i, r: (i, 0, 0)),
            pl.BlockSpec((1, NEMB, CDIM), lambda i, r: (i, 0, 0)),
            pl.BlockSpec((1, 1, NEMB), lambda i, r: (i, 0, 0)),
        ],
        out_specs=[
            pl.BlockSpec((TILE, CDIM), lambda i, r: (r, i)),
            pl.BlockSpec((1, 1, 1, TILE), lambda i, r: (i, r, 0, 0)),
            pl.BlockSpec((1, 1), lambda i, r: (0, 0)),
            pl.BlockSpec((1, NCB), lambda i, r: (0, 0)),
        ],
        out_shape=[
            jax.ShapeDtypeStruct((ROWS, NCB * CDIM), jnp.float32),
            jax.ShapeDtypeStruct((NCB, NT, 1, TILE), jnp.int32),
            jax.ShapeDtypeStruct((1, 1), jnp.float32),
            jax.ShapeDtypeStruct((1, NCB), jnp.float32),
        ],
        scratch_shapes=[
            pltpu.VMEM((1, NEMB), jnp.float32),
            pltpu.SMEM((1,), jnp.float32),
        ],
        compiler_params=pltpu.CompilerParams(
            dimension_semantics=("arbitrary", "arbitrary")),
    )(zf, et, e, e2)

    out = pl.pallas_call(
        _stage_c,
        grid=(B,),
        in_specs=[
            pl.BlockSpec((1, T, LATENT), lambda b: (b, 0, 0)),
            pl.BlockSpec((FEAT, LATENT), lambda b: (0, 0)),
        ],
        out_specs=pl.BlockSpec((1, FEAT, T), lambda b: (b, 0, 0)),
        out_shape=jax.ShapeDtypeStruct((B, FEAT, T), jnp.float32),
        compiler_params=pltpu.CompilerParams(
            dimension_semantics=("arbitrary",)),
    )(qf.reshape(B, T, LATENT), w2)

    return out, kidx, commit, ppls


def kernel(inputs, W1, W2, embeddings):
    b, c, t, _ = inputs.shape
    out, kidx, commit, ppls = _run(inputs[..., 0], W1, W2, embeddings)
    inds = kidx.reshape(NCB, b, t // CF)
    return out[..., None], commit[0, 0], ppls[0], inds


# speedup vs baseline: 2.1848x; 1.2057x over previous
import jax
import jax.numpy as jnp
from jax import lax
from jax.experimental import pallas as pl
from jax.experimental.pallas import tpu as pltpu

FEAT = 512
LATENT = 256
CF = 4
NCB = 4
NEMB = 1024
CDIM = 256
ALPHA = -5.0
B = 32
T = 800
ROWS = (B * T) // NCB
TILE = 640
NT = ROWS // TILE
BIG = 2 ** 30


def _stage_a(x_ref, w1_ref, z_ref):
    zt = lax.dot_general(w1_ref[...], x_ref[0].astype(jnp.bfloat16),
                         (((1,), (0,)), ((), ())),
                         preferred_element_type=jnp.float32)
    z_ref[0] = jnp.transpose(zt).reshape(T // CF, NCB * LATENT)


def _stage_b(z_ref, et_ref, e_ref, e2_ref,
             q_ref, kidx_ref, commit_ref, ppls_ref,
             counts_ref, acc_ref):
    i = pl.program_id(0)
    r = pl.program_id(1)

    z = z_ref[...]
    xe = lax.dot_general(z.astype(jnp.bfloat16), et_ref[0],
                         (((1,), (0,)), ((), ())),
                         preferred_element_type=jnp.float32)
    x2 = jnp.sum(z * z, axis=1, keepdims=True)
    d = e2_ref[0] + x2 - 2.0 * xe
    dm = ALPHA * d
    m = jnp.max(dm, axis=1, keepdims=True)
    iota = lax.broadcasted_iota(jnp.int32, (TILE, NEMB), 1)
    k2 = jnp.min(jnp.where(dm == m, iota, BIG), axis=1, keepdims=True)

    hard = (iota == k2).astype(jnp.bfloat16)
    q = lax.dot_general(hard, e_ref[0],
                        (((1,), (0,)), ((), ())),
                        preferred_element_type=jnp.float32)
    q_ref[...] = q
    kidx_ref[...] = jnp.transpose(k2).reshape(1, 1, 1, TILE)

    ones = jnp.full((8, TILE), jnp.bfloat16(1))
    cnt8 = lax.dot_general(ones, hard, (((1,), (0,)), ((), ())),
                           preferred_element_type=jnp.float32)
    cnt = cnt8[0:1]

    @pl.when(r == 0)
    def _():
        counts_ref[...] = cnt

    @pl.when(r > 0)
    def _():
        counts_ref[...] = counts_ref[...] + cnt

    part = jnp.sum((z - q) ** 2)

    @pl.when((i == 0) & (r == 0))
    def _():
        acc_ref[0] = part

    @pl.when((i > 0) | (r > 0))
    def _():
        acc_ref[0] = acc_ref[0] + part

    @pl.when((i == NCB - 1) & (r == NT - 1))
    def _():
        commit_ref[...] = jnp.full((1, 1), acc_ref[0] / (ROWS * CDIM),
                                   jnp.float32)

    @pl.when(r == NT - 1)
    def _():
        probs = counts_ref[...] / float(ROWS)
        ent = -jnp.sum(probs * jnp.log2(probs + 1e-10))
        lane4 = lax.broadcasted_iota(jnp.int32, (1, NCB), 1)
        contrib = jnp.where(lane4 == i, ent, 0.0)
        prev = jnp.where(i == 0, jnp.zeros((1, NCB), jnp.float32),
                         ppls_ref[...])
        ppls_ref[...] = prev + contrib


def _stage_c(q_ref, w2_ref, out_ref):
    qrows = q_ref[0].reshape(T, LATENT)
    qt = jnp.transpose(qrows).astype(jnp.bfloat16)
    out_ref[0] = lax.dot_general(w2_ref[...], qt,
                                 (((1,), (0,)), ((), ())),
                                 preferred_element_type=jnp.float32)


@jax.jit
def _run(x, W1, W2, embeddings):
    w1 = W1.astype(jnp.bfloat16)
    w2 = W2.astype(jnp.bfloat16)
    et = jnp.transpose(embeddings, (0, 2, 1)).astype(jnp.bfloat16)
    e = embeddings.astype(jnp.bfloat16)
    e2 = jnp.sum(embeddings ** 2, axis=2)[:, None, :]

    z = pl.pallas_call(
        _stage_a,
        grid=(B,),
        in_specs=[
            pl.BlockSpec((1, FEAT, T), lambda b: (b, 0, 0)),
            pl.BlockSpec((LATENT, FEAT), lambda b: (0, 0)),
        ],
        out_specs=pl.BlockSpec((1, T // CF, NCB * CDIM), lambda b: (b, 0, 0)),
        out_shape=jax.ShapeDtypeStruct((B, T // CF, NCB * CDIM), jnp.float32),
        compiler_params=pltpu.CompilerParams(
            dimension_semantics=("arbitrary",)),
    )(x, w1)

    zf = z.reshape(ROWS, NCB * CDIM)

    qf, kidx, commit, ppls = pl.pallas_call(
        _stage_b,
        grid=(NCB, NT),
        in_specs=[
            pl.BlockSpec((TILE, CDIM), lambda i, r: (r, i)),
            pl.BlockSpec((1, CDIM, NEMB), lambda i, r: (i, 0, 0)),
            pl.BlockSpec((1, NEMB, CDIM), lambda i, r: (i, 0, 0)),
            pl.BlockSpec((1, 1, NEMB), lambda i, r: (i, 0, 0)),
        ],
        out_specs=[
            pl.BlockSpec((TILE, CDIM), lambda i, r: (r, i)),
            pl.BlockSpec((1, 1, 1, TILE), lambda i, r: (i, r, 0, 0)),
            pl.BlockSpec((1, 1), lambda i, r: (0, 0)),
            pl.BlockSpec((1, NCB), lambda i, r: (0, 0)),
        ],
        out_shape=[
            jax.ShapeDtypeStruct((ROWS, NCB * CDIM), jnp.float32),
            jax.ShapeDtypeStruct((NCB, NT, 1, TILE), jnp.int32),
            jax.ShapeDtypeStruct((1, 1), jnp.float32),
            jax.ShapeDtypeStruct((1, NCB), jnp.float32),
        ],
        scratch_shapes=[
            pltpu.VMEM((1, NEMB), jnp.float32),
            pltpu.SMEM((1,), jnp.float32),
        ],
        compiler_params=pltpu.CompilerParams(
            dimension_semantics=("arbitrary", "arbitrary")),
    )(zf, et, e, e2)

    out = pl.pallas_call(
        _stage_c,
        grid=(B,),
        in_specs=[
            pl.BlockSpec((1, T // CF, NCB * CDIM), lambda b: (b, 0, 0)),
            pl.BlockSpec((FEAT, LATENT), lambda b: (0, 0)),
        ],
        out_specs=pl.BlockSpec((1, FEAT, T), lambda b: (b, 0, 0)),
        out_shape=jax.ShapeDtypeStruct((B, FEAT, T), jnp.float32),
        compiler_params=pltpu.CompilerParams(
            dimension_semantics=("arbitrary",)),
    )(qf.reshape(B, T // CF, NCB * CDIM), w2)

    return out, kidx, commit, ppls


def kernel(inputs, W1, W2, embeddings):
    b, c, t, _ = inputs.shape
    out, kidx, commit, ppls = _run(inputs[..., 0], W1, W2, embeddings)
    inds = kidx.reshape(NCB, b, t // CF)
    return out[..., None], commit[0, 0], ppls[0], inds


# speedup vs baseline: 2.2878x; 1.0472x over previous
import jax
import jax.numpy as jnp
from jax import lax
from jax.experimental import pallas as pl
from jax.experimental.pallas import tpu as pltpu

FEAT = 512
LATENT = 256
CF = 4
NCB = 4
NEMB = 1024
CDIM = 256
ALPHA = -5.0
B = 32
T = 800
ROWS = (B * T) // NCB
TILE = 640
NT = ROWS // TILE
BIG = 2 ** 30


def _stage_a(x_ref, w1t_ref, z_ref):
    z = lax.dot_general(x_ref[0].astype(jnp.bfloat16), w1t_ref[...],
                        (((1,), (0,)), ((), ())),
                        preferred_element_type=jnp.float32)
    z_ref[0] = z.reshape(T // CF, NCB * LATENT)


def _stage_b(z_ref, et_ref, e_ref, e2_ref,
             q_ref, kidx_ref, commit_ref, ppls_ref,
             counts_ref, acc_ref):
    i = pl.program_id(0)
    r = pl.program_id(1)

    z = z_ref[...]
    xe = lax.dot_general(z.astype(jnp.bfloat16), et_ref[0],
                         (((1,), (0,)), ((), ())),
                         preferred_element_type=jnp.float32)
    x2 = jnp.sum(z * z, axis=1, keepdims=True)
    d = e2_ref[0] + x2 - 2.0 * xe
    dm = ALPHA * d
    m = jnp.max(dm, axis=1, keepdims=True)
    iota = lax.broadcasted_iota(jnp.int32, (TILE, NEMB), 1)
    k2 = jnp.min(jnp.where(dm == m, iota, BIG), axis=1, keepdims=True)

    hard = (iota == k2).astype(jnp.bfloat16)
    q = lax.dot_general(hard, e_ref[0],
                        (((1,), (0,)), ((), ())),
                        preferred_element_type=jnp.float32)
    q_ref[...] = q
    kidx_ref[...] = jnp.transpose(k2).reshape(1, 1, 1, TILE)

    ones = jnp.full((8, TILE), jnp.bfloat16(1))
    cnt8 = lax.dot_general(ones, hard, (((1,), (0,)), ((), ())),
                           preferred_element_type=jnp.float32)
    cnt = cnt8[0:1]

    @pl.when(r == 0)
    def _():
        counts_ref[...] = cnt

    @pl.when(r > 0)
    def _():
        counts_ref[...] = counts_ref[...] + cnt

    part = jnp.sum((z - q) ** 2)

    @pl.when((i == 0) & (r == 0))
    def _():
        acc_ref[0] = part

    @pl.when((i > 0) | (r > 0))
    def _():
        acc_ref[0] = acc_ref[0] + part

    @pl.when((i == NCB - 1) & (r == NT - 1))
    def _():
        commit_ref[...] = jnp.full((1, 1), acc_ref[0] / (ROWS * CDIM),
                                   jnp.float32)

    @pl.when(r == NT - 1)
    def _():
        probs = counts_ref[...] / float(ROWS)
        ent = -jnp.sum(probs * jnp.log2(probs + 1e-10))
        lane4 = lax.broadcasted_iota(jnp.int32, (1, NCB), 1)
        contrib = jnp.where(lane4 == i, ent, 0.0)
        prev = jnp.where(i == 0, jnp.zeros((1, NCB), jnp.float32),
                         ppls_ref[...])
        ppls_ref[...] = prev + contrib


def _stage_c(q_ref, w2t_ref, out_ref):
    qrows = q_ref[0].reshape(T, LATENT).astype(jnp.bfloat16)
    out_ref[0] = lax.dot_general(qrows, w2t_ref[...],
                                 (((1,), (0,)), ((), ())),
                                 preferred_element_type=jnp.float32)


@jax.jit
def _run(x, W1, W2, embeddings):
    w1t = W1.T.astype(jnp.bfloat16)
    w2t = W2.T.astype(jnp.bfloat16)
    et = jnp.transpose(embeddings, (0, 2, 1)).astype(jnp.bfloat16)
    e = embeddings.astype(jnp.bfloat16)
    e2 = jnp.sum(embeddings ** 2, axis=2)[:, None, :]

    z = pl.pallas_call(
        _stage_a,
        grid=(B,),
        in_specs=[
            pl.BlockSpec((1, T, FEAT), lambda b: (b, 0, 0)),
            pl.BlockSpec((FEAT, LATENT), lambda b: (0, 0)),
        ],
        out_specs=pl.BlockSpec((1, T // CF, NCB * CDIM), lambda b: (b, 0, 0)),
        out_shape=jax.ShapeDtypeStruct((B, T // CF, NCB * CDIM), jnp.float32),
        compiler_params=pltpu.CompilerParams(
            dimension_semantics=("arbitrary",)),
    )(x, w1t)

    zf = z.reshape(ROWS, NCB * CDIM)

    qf, kidx, commit, ppls = pl.pallas_call(
        _stage_b,
        grid=(NCB, NT),
        in_specs=[
            pl.BlockSpec((TILE, CDIM), lambda i, r: (r, i)),
            pl.BlockSpec((1, CDIM, NEMB), lambda i, r: (i, 0, 0)),
            pl.BlockSpec((1, NEMB, CDIM), lambda i, r: (i, 0, 0)),
            pl.BlockSpec((1, 1, NEMB), lambda i, r: (i, 0, 0)),
        ],
        out_specs=[
            pl.BlockSpec((TILE, CDIM), lambda i, r: (r, i)),
            pl.BlockSpec((1, 1, 1, TILE), lambda i, r: (i, r, 0, 0)),
            pl.BlockSpec((1, 1), lambda i, r: (0, 0)),
            pl.BlockSpec((1, NCB), lambda i, r: (0, 0)),
        ],
        out_shape=[
            jax.ShapeDtypeStruct((ROWS, NCB * CDIM), jnp.float32),
            jax.ShapeDtypeStruct((NCB, NT, 1, TILE), jnp.int32),
            jax.ShapeDtypeStruct((1, 1), jnp.float32),
            jax.ShapeDtypeStruct((1, NCB), jnp.float32),
        ],
        scratch_shapes=[
            pltpu.VMEM((1, NEMB), jnp.float32),
            pltpu.SMEM((1,), jnp.float32),
        ],
        compiler_params=pltpu.CompilerParams(
            dimension_semantics=("arbitrary", "arbitrary")),
    )(zf, et, e, e2)

    out = pl.pallas_call(
        _stage_c,
        grid=(B,),
        in_specs=[
            pl.BlockSpec((1, T // CF, NCB * CDIM), lambda b: (b, 0, 0)),
            pl.BlockSpec((LATENT, FEAT), lambda b: (0, 0)),
        ],
        out_specs=pl.BlockSpec((1, T, FEAT), lambda b: (b, 0, 0)),
        out_shape=jax.ShapeDtypeStruct((B, T, FEAT), jnp.float32),
        compiler_params=pltpu.CompilerParams(
            dimension_semantics=("arbitrary",)),
    )(qf.reshape(B, T // CF, NCB * CDIM), w2t)

    return out, kidx, commit, ppls


def kernel(inputs, W1, W2, embeddings):
    b, c, t, _ = inputs.shape
    xt = jnp.transpose(inputs[..., 0], (0, 2, 1))
    out3, kidx, commit, ppls = _run(xt, W1, W2, embeddings)
    out = jnp.transpose(out3, (0, 2, 1))[..., None]
    inds = kidx.reshape(NCB, b, t // CF)
    return out, commit[0, 0], ppls[0], inds
